# baseline (speedup 1.0000x reference)
import functools

import jax
import jax.numpy as jnp
import numpy as np
from jax import lax
from jax.experimental import pallas as pl
from jax.experimental.pallas import tpu as pltpu
from jax.experimental.pallas import tpu_sc as plsc

_N = 10000
_E = 320000
_D = 128
_K = 6

_NC = 2
_NS = 16
_NW = _NC * _NS
_NP = 10240
_RPS = _NP // _NS
_CH = 128
_NPH = 2
_NCHP = 40
_NCH = _NPH * _NCHP
_EPAD = _NW * _NCH * _CH


def _cheb_coeff_mat(k):
    j = np.arange(k + 1)
    xj = np.cos((k - j + 0.5) * np.pi / (k + 1))
    t = np.zeros((k + 1, k + 1), dtype=np.float64)
    t[0, :] = 1.0
    t[1, :] = xj
    for i in range(2, k + 1):
        t[i, :] = 2.0 * xj * t[i - 1, :] - t[i - 2, :]
    return (2.0 / (k + 1)) * t


_TMAT = _cheb_coeff_mat(_K).astype(np.float32)

_MESH = plsc.VectorSubcoreMesh(
    core_axis_name="c", subcore_axis_name="s", num_cores=_NC, num_subcores=_NS
)


def _deg_body(row_hbm, zcol_hbm, out_hbm, row_v, ones_v, deg_sh, sem):
    cid = lax.axis_index("c")
    sid = lax.axis_index("s")
    wid = cid * _NS + sid
    sl = pl.ds(sid * _RPS, _RPS)
    pltpu.sync_copy(zcol_hbm.at[sl], deg_sh.at[sl])
    for k in range(_CH // 16):
        ones_v[pl.ds(k * 16, 16)] = jnp.full((16,), 1.0, jnp.float32)
    plsc.subcore_barrier()

    for p in range(_NPH):
        pltpu.sync_copy(row_hbm.at[wid, p], row_v)

        def fire(j, carry):
            pltpu.async_copy(ones_v, deg_sh.at[row_v.at[j]], sem, add=True)
            return carry

        lax.fori_loop(0, _NCHP, fire, 0)

        def drain(j, carry):
            pltpu.make_async_copy(ones_v, deg_sh.at[row_v.at[j]], sem).wait()
            return carry

        lax.fori_loop(0, _NCHP, drain, 0)
    plsc.subcore_barrier()
    pltpu.sync_copy(deg_sh.at[sl], out_hbm.at[cid, sl])


_deg_call = pl.kernel(
    _deg_body,
    out_type=jax.ShapeDtypeStruct((_NC, _NP), jnp.float32),
    mesh=_MESH,
    scratch_types=[
        pltpu.VMEM((_NCHP, _CH), jnp.int32),
        pltpu.VMEM((_CH,), jnp.float32),
        pltpu.VMEM_SHARED((_NP,), jnp.float32),
        pltpu.SemaphoreType.DMA,
    ],
)


def _zero_buf(buf, rows):
    def zbody(r, carry):
        for k in range(_D // 16):
            buf[r, pl.ds(k * 16, 16)] = jnp.full((16,), 0.0, jnp.float32)
        return carry

    lax.fori_loop(0, rows, zbody, 0)


def _prop_body(u_hbm, row_hbm, col_hbm, ztile_hbm, out_hbm,
               row_v, col_v, buf0, buf1, s_sh, sem0, sem1):
    cid = lax.axis_index("c")
    sid = lax.axis_index("s")
    wid = cid * _NS + sid
    pltpu.sync_copy(ztile_hbm, buf0)
    for r0 in range(0, _RPS, _CH):
        pltpu.sync_copy(buf0, s_sh.at[pl.ds(sid * _RPS + r0, _CH)])
    plsc.subcore_barrier()

    for p in range(_NPH):
        pltpu.sync_copy(row_hbm.at[wid, p], row_v)
        pltpu.sync_copy(col_hbm.at[wid, p], col_v)

        pltpu.async_copy(u_hbm.at[row_v.at[0]], buf0, sem0)
        pltpu.async_copy(u_hbm.at[row_v.at[1]], buf1, sem1)

        def body(i, carry):
            j = i * 2
            for b, (buf, sem) in enumerate(((buf0, sem0), (buf1, sem1))):
                jb = j + b
                pltpu.make_async_copy(u_hbm.at[row_v.at[jb]], buf, sem).wait()
                pltpu.sync_copy(buf, s_sh.at[col_v.at[jb]], add=True)

                @pl.when(jb + 2 < _NCHP)
                def _():
                    pltpu.async_copy(u_hbm.at[row_v.at[jb + 2]], buf, sem)
            return carry

        lax.fori_loop(0, _NCHP // 2, body, 0)
    plsc.subcore_barrier()
    sl = pl.ds(sid * _RPS, _RPS)
    pltpu.sync_copy(s_sh.at[sl], out_hbm.at[cid, sl])


_prop_call = pl.kernel(
    _prop_body,
    out_type=jax.ShapeDtypeStruct((_NC, _NP, _D), jnp.float32),
    mesh=_MESH,
    scratch_types=[
        pltpu.VMEM((_NCHP, _CH), jnp.int32),
        pltpu.VMEM((_NCHP, _CH), jnp.int32),
        pltpu.VMEM((_CH, _D), jnp.float32),
        pltpu.VMEM((_CH, _D), jnp.float32),
        pltpu.VMEM_SHARED((_NP, _D), jnp.float32),
        pltpu.SemaphoreType.DMA,
        pltpu.SemaphoreType.DMA,
    ],
)


_BR = 1024


def _mlp_init_body(xp_ref, w1_ref, b1_ref, w2_ref, b2_ref,
                   deg0_ref, deg1_ref, x_ref, u_ref, dis1_ref):
    h = jnp.maximum(
        jnp.dot(xp_ref[...], w1_ref[...], preferred_element_type=jnp.float32)
        + b1_ref[...], 0.0)
    x = (jnp.dot(h, w2_ref[...], preferred_element_type=jnp.float32)
         + b2_ref[...])
    deg = deg0_ref[...] + deg1_ref[...]
    dis = jnp.where(deg > 0.0, lax.rsqrt(jnp.maximum(deg, 1e-30)), 0.0)
    x_ref[...] = x
    u_ref[...] = jnp.broadcast_to(dis, x.shape) * x
    dis1_ref[...] = dis


def _update_first_body(c_ref, s0_ref, s1_ref, dis1_ref, txold_ref,
                       tx_ref, u_ref, out_ref):
    dis2 = jnp.broadcast_to(dis1_ref[...], s0_ref.shape)
    tx = -dis2 * (s0_ref[...] + s1_ref[...])
    tx_ref[...] = tx
    u_ref[...] = dis2 * tx
    out_ref[...] = c_ref[0, 0] * txold_ref[...] + c_ref[0, 1] * tx


def _update_rest_body(c_ref, s0_ref, s1_ref, dis1_ref, txold_ref,
                      outold_ref, tx_ref, u_ref, out_ref):
    dis2 = jnp.broadcast_to(dis1_ref[...], s0_ref.shape)
    tx = -2.0 * dis2 * (s0_ref[...] + s1_ref[...]) - txold_ref[...]
    tx_ref[...] = tx
    u_ref[...] = dis2 * tx
    out_ref[...] = outold_ref[...] + c_ref[0, 0] * tx


def _update_last_body(c_ref, s0_ref, s1_ref, dis1_ref, txold_ref,
                      outold_ref, out_ref):
    dis2 = jnp.broadcast_to(dis1_ref[...], s0_ref.shape)
    tx = -2.0 * dis2 * (s0_ref[...] + s1_ref[...]) - txold_ref[...]
    out_ref[...] = outold_ref[...] + c_ref[0, 0] * tx


def _row_spec(i):
    return (i, 0)


def _fixed_spec(i):
    return (0, 0)


_GRID = (_NP // _BR,)
_f32 = jnp.float32


def _mk_mlp_init():
    return pl.pallas_call(
        _mlp_init_body,
        grid=_GRID,
        in_specs=[
            pl.BlockSpec((_BR, _D), _row_spec),
            pl.BlockSpec((_D, _D), _fixed_spec),
            pl.BlockSpec((1, _D), _fixed_spec),
            pl.BlockSpec((_D, _D), _fixed_spec),
            pl.BlockSpec((1, _D), _fixed_spec),
            pl.BlockSpec((_BR, 1), _row_spec),
            pl.BlockSpec((_BR, 1), _row_spec),
        ],
        out_specs=[pl.BlockSpec((_BR, _D), _row_spec)] * 2
        + [pl.BlockSpec((_BR, 1), _row_spec)],
        out_shape=[jax.ShapeDtypeStruct((_NP, _D), _f32)] * 2
        + [jax.ShapeDtypeStruct((_NP, 1), _f32)],
    )


_D_SPEC = pl.BlockSpec((_BR, _D), _row_spec)
_C_SPEC = pl.BlockSpec((_BR, 1), _row_spec)
_S_SPEC = pl.BlockSpec(memory_space=pltpu.SMEM)
_D_SHAPE = jax.ShapeDtypeStruct((_NP, _D), _f32)

_mlp_init = _mk_mlp_init()
_update_first = pl.pallas_call(
    _update_first_body, grid=_GRID,
    in_specs=[_S_SPEC, _D_SPEC, _D_SPEC, _C_SPEC, _D_SPEC],
    out_specs=[_D_SPEC] * 3, out_shape=[_D_SHAPE] * 3)
_update_rest = pl.pallas_call(
    _update_rest_body, grid=_GRID,
    in_specs=[_S_SPEC, _D_SPEC, _D_SPEC, _C_SPEC, _D_SPEC, _D_SPEC],
    out_specs=[_D_SPEC] * 3, out_shape=[_D_SHAPE] * 3)
_update_last = pl.pallas_call(
    _update_last_body, grid=_GRID,
    in_specs=[_S_SPEC, _D_SPEC, _D_SPEC, _C_SPEC, _D_SPEC, _D_SPEC],
    out_specs=_D_SPEC, out_shape=_D_SHAPE)


def kernel(X, edge_index, W1, b1, W2, b2, temp):
    row = edge_index[0].astype(jnp.int32)
    col = edge_index[1].astype(jnp.int32)
    npad = _EPAD - _E
    pad_ar = jnp.arange(npad, dtype=jnp.int32)
    dump = _N + (pad_ar % (_NP - _N))
    realpad = pad_ar % _N
    shp = (_NW, _NPH, _NCHP, _CH)
    rowp = jnp.concatenate([row, realpad]).reshape(shp)
    rowp_deg = jnp.concatenate([row, dump]).reshape(shp)
    colp = jnp.concatenate([col, dump]).reshape(shp)

    coe = jnp.asarray(_TMAT) @ jax.nn.relu(temp.astype(jnp.float32))
    xpad = jnp.pad(X.astype(jnp.float32), ((0, _NP - _N), (0, 0)))
    zcol = jnp.zeros((_NP,), _f32)
    ztile = jnp.zeros((_CH, _D), _f32)

    deg01 = _deg_call(rowp_deg, zcol)
    deg0 = deg01[0].reshape(_NP, 1)
    deg1 = deg01[1].reshape(_NP, 1)

    x0, u, dis1 = _mlp_init(
        xpad, W1.astype(_f32), b1.reshape(1, _D).astype(_f32),
        W2.astype(_f32), b2.reshape(1, _D).astype(_f32), deg0, deg1)

    tx_prev2 = x0
    tx_prev = x0
    outa = None
    for j in range(1, _K + 1):
        s01 = _prop_call(u, rowp, colp, ztile)
        if j == 1:
            cc = jnp.stack([coe[0] * 0.5, coe[1]]).reshape(1, 2)
            tx_new, u, outa = _update_first(cc, s01[0], s01[1], dis1, tx_prev2)
        elif j == _K:
            cj = coe[j].reshape(1, 1)
            outa = _update_last(cj, s01[0], s01[1], dis1, tx_prev2, outa)
            break
        else:
            cj = coe[j].reshape(1, 1)
            tx_new, u, outa = _update_rest(
                cj, s01[0], s01[1], dis1, tx_prev2, outa)
        tx_prev2 = tx_prev
        tx_prev = tx_new

    return outa[:_N]


# speedup vs baseline: 1.0346x; 1.0346x over previous
import functools

import jax
import jax.numpy as jnp
import numpy as np
from jax import lax
from jax.experimental import pallas as pl
from jax.experimental.pallas import tpu as pltpu
from jax.experimental.pallas import tpu_sc as plsc

_N = 10000
_E = 320000
_D = 128
_K = 6

_NC = 2
_NS = 16
_NW = _NC * _NS
_NP = 10240
_RPS = _NP // _NS
_CH = 128
_NPH = 2
_NCHP = 40
_NCH = _NPH * _NCHP
_EPAD = _NW * _NCH * _CH


def _cheb_coeff_mat(k):
    j = np.arange(k + 1)
    xj = np.cos((k - j + 0.5) * np.pi / (k + 1))
    t = np.zeros((k + 1, k + 1), dtype=np.float64)
    t[0, :] = 1.0
    t[1, :] = xj
    for i in range(2, k + 1):
        t[i, :] = 2.0 * xj * t[i - 1, :] - t[i - 2, :]
    return (2.0 / (k + 1)) * t


_TMAT = _cheb_coeff_mat(_K).astype(np.float32)

_MESH = plsc.VectorSubcoreMesh(
    core_axis_name="c", subcore_axis_name="s", num_cores=_NC, num_subcores=_NS
)


def _deg_body(row_hbm, zcol_hbm, out_hbm, row_v, ones_v, deg_sh, sem):
    cid = lax.axis_index("c")
    sid = lax.axis_index("s")
    wid = cid * _NS + sid
    sl = pl.ds(sid * _RPS, _RPS)
    pltpu.sync_copy(zcol_hbm.at[sl], deg_sh.at[sl])
    for k in range(_CH // 16):
        ones_v[pl.ds(k * 16, 16)] = jnp.full((16,), 1.0, jnp.float32)
    plsc.subcore_barrier()

    for p in range(_NPH):
        pltpu.sync_copy(row_hbm.at[wid, p], row_v)

        def fire(j, carry):
            pltpu.async_copy(ones_v, deg_sh.at[row_v.at[j]], sem, add=True)
            return carry

        lax.fori_loop(0, _NCHP, fire, 0)

        def drain(j, carry):
            pltpu.make_async_copy(ones_v, deg_sh.at[row_v.at[j]], sem).wait()
            return carry

        lax.fori_loop(0, _NCHP, drain, 0)
    plsc.subcore_barrier()
    pltpu.sync_copy(deg_sh.at[sl], out_hbm.at[cid, sl])


_deg_call = pl.kernel(
    _deg_body,
    out_type=jax.ShapeDtypeStruct((_NC, _NP), jnp.float32),
    mesh=_MESH,
    scratch_types=[
        pltpu.VMEM((_NCHP, _CH), jnp.int32),
        pltpu.VMEM((_CH,), jnp.float32),
        pltpu.VMEM_SHARED((_NP,), jnp.float32),
        pltpu.SemaphoreType.DMA,
    ],
)


def _zero_buf(buf, rows):
    def zbody(r, carry):
        for k in range(_D // 16):
            buf[r, pl.ds(k * 16, 16)] = jnp.full((16,), 0.0, jnp.float32)
        return carry

    lax.fori_loop(0, rows, zbody, 0)


def _prop_body(u_hbm, row_hbm, col_hbm, out_hbm,
               row_v, col_v, buf0, buf1, s_sh, sem0, sem1):
    cid = lax.axis_index("c")
    sid = lax.axis_index("s")
    wid = cid * _NS + sid
    _zero_buf(buf0, _CH)
    for r0 in range(0, _RPS, _CH):
        pltpu.sync_copy(buf0, s_sh.at[pl.ds(sid * _RPS + r0, _CH)])
    plsc.subcore_barrier()

    for p in range(_NPH):
        pltpu.sync_copy(row_hbm.at[wid, p], row_v)
        pltpu.sync_copy(col_hbm.at[wid, p], col_v)

        pltpu.async_copy(u_hbm.at[row_v.at[0]], buf0, sem0)
        pltpu.async_copy(u_hbm.at[row_v.at[1]], buf1, sem1)

        def body(i, carry):
            j = i * 2
            for b, (buf, sem) in enumerate(((buf0, sem0), (buf1, sem1))):
                jb = j + b
                pltpu.make_async_copy(u_hbm.at[row_v.at[jb]], buf, sem).wait()
                pltpu.sync_copy(buf, s_sh.at[col_v.at[jb]], add=True)

                @pl.when(jb + 2 < _NCHP)
                def _():
                    pltpu.async_copy(u_hbm.at[row_v.at[jb + 2]], buf, sem)
            return carry

        lax.fori_loop(0, _NCHP // 2, body, 0)
    plsc.subcore_barrier()
    sl = pl.ds(sid * _RPS, _RPS)
    pltpu.sync_copy(s_sh.at[sl], out_hbm.at[cid, sl])


_prop_call = pl.kernel(
    _prop_body,
    out_type=jax.ShapeDtypeStruct((_NC, _NP, _D), jnp.float32),
    mesh=_MESH,
    scratch_types=[
        pltpu.VMEM((_NCHP, _CH), jnp.int32),
        pltpu.VMEM((_NCHP, _CH), jnp.int32),
        pltpu.VMEM((_CH, _D), jnp.float32),
        pltpu.VMEM((_CH, _D), jnp.float32),
        pltpu.VMEM_SHARED((_NP, _D), jnp.float32),
        pltpu.SemaphoreType.DMA,
        pltpu.SemaphoreType.DMA,
    ],
)


_BR = 2048


def _mlp_init_body(xp_ref, w1_ref, b1_ref, w2_ref, b2_ref,
                   deg0_ref, deg1_ref, x_ref, u_ref, dis1_ref):
    h = jnp.maximum(
        jnp.dot(xp_ref[...], w1_ref[...], preferred_element_type=jnp.float32)
        + b1_ref[...], 0.0)
    x = (jnp.dot(h, w2_ref[...], preferred_element_type=jnp.float32)
         + b2_ref[...])
    deg = deg0_ref[...] + deg1_ref[...]
    dis = jnp.where(deg > 0.0, lax.rsqrt(jnp.maximum(deg, 1e-30)), 0.0)
    x_ref[...] = x
    u_ref[...] = jnp.broadcast_to(dis, x.shape) * x
    dis1_ref[...] = dis


def _update_first_body(c_ref, s0_ref, s1_ref, dis1_ref, txold_ref,
                       tx_ref, u_ref, out_ref):
    dis2 = jnp.broadcast_to(dis1_ref[...], s0_ref.shape)
    tx = -dis2 * (s0_ref[...] + s1_ref[...])
    tx_ref[...] = tx
    u_ref[...] = dis2 * tx
    out_ref[...] = c_ref[0, 0] * txold_ref[...] + c_ref[0, 1] * tx


def _update_rest_body(c_ref, s0_ref, s1_ref, dis1_ref, txold_ref,
                      outold_ref, tx_ref, u_ref, out_ref):
    dis2 = jnp.broadcast_to(dis1_ref[...], s0_ref.shape)
    tx = -2.0 * dis2 * (s0_ref[...] + s1_ref[...]) - txold_ref[...]
    tx_ref[...] = tx
    u_ref[...] = dis2 * tx
    out_ref[...] = outold_ref[...] + c_ref[0, 0] * tx


def _update_last_body(c_ref, s0_ref, s1_ref, dis1_ref, txold_ref,
                      outold_ref, out_ref):
    dis2 = jnp.broadcast_to(dis1_ref[...], s0_ref.shape)
    tx = -2.0 * dis2 * (s0_ref[...] + s1_ref[...]) - txold_ref[...]
    out_ref[...] = outold_ref[...] + c_ref[0, 0] * tx


def _row_spec(i):
    return (i, 0)


def _fixed_spec(i):
    return (0, 0)


_GRID = (_NP // _BR,)
_f32 = jnp.float32


def _mk_mlp_init():
    return pl.pallas_call(
        _mlp_init_body,
        grid=_GRID,
        in_specs=[
            pl.BlockSpec((_BR, _D), _row_spec),
            pl.BlockSpec((_D, _D), _fixed_spec),
            pl.BlockSpec((1, _D), _fixed_spec),
            pl.BlockSpec((_D, _D), _fixed_spec),
            pl.BlockSpec((1, _D), _fixed_spec),
            pl.BlockSpec((_BR, 1), _row_spec),
            pl.BlockSpec((_BR, 1), _row_spec),
        ],
        out_specs=[pl.BlockSpec((_BR, _D), _row_spec)] * 2
        + [pl.BlockSpec((_BR, 1), _row_spec)],
        out_shape=[jax.ShapeDtypeStruct((_NP, _D), _f32)] * 2
        + [jax.ShapeDtypeStruct((_NP, 1), _f32)],
    )


_D_SPEC = pl.BlockSpec((_BR, _D), _row_spec)
_C_SPEC = pl.BlockSpec((_BR, 1), _row_spec)
_S_SPEC = pl.BlockSpec(memory_space=pltpu.SMEM)
_D_SHAPE = jax.ShapeDtypeStruct((_NP, _D), _f32)

_mlp_init = _mk_mlp_init()
_update_first = pl.pallas_call(
    _update_first_body, grid=_GRID,
    in_specs=[_S_SPEC, _D_SPEC, _D_SPEC, _C_SPEC, _D_SPEC],
    out_specs=[_D_SPEC] * 3, out_shape=[_D_SHAPE] * 3)
_update_rest = pl.pallas_call(
    _update_rest_body, grid=_GRID,
    in_specs=[_S_SPEC, _D_SPEC, _D_SPEC, _C_SPEC, _D_SPEC, _D_SPEC],
    out_specs=[_D_SPEC] * 3, out_shape=[_D_SHAPE] * 3)
_update_last = pl.pallas_call(
    _update_last_body, grid=_GRID,
    in_specs=[_S_SPEC, _D_SPEC, _D_SPEC, _C_SPEC, _D_SPEC, _D_SPEC],
    out_specs=_D_SPEC, out_shape=_D_SHAPE)


def kernel(X, edge_index, W1, b1, W2, b2, temp):
    row = edge_index[0].astype(jnp.int32)
    col = edge_index[1].astype(jnp.int32)
    npad = _EPAD - _E
    pad_ar = jnp.arange(npad, dtype=jnp.int32)
    dump = _N + (pad_ar % (_NP - _N))
    realpad = pad_ar % _N
    shp = (_NW, _NPH, _NCHP, _CH)
    rowp = jnp.concatenate([row, realpad]).reshape(shp)
    rowp_deg = jnp.concatenate([row, dump]).reshape(shp)
    colp = jnp.concatenate([col, dump]).reshape(shp)

    coe = jnp.asarray(_TMAT) @ jax.nn.relu(temp.astype(jnp.float32))
    xpad = jnp.pad(X.astype(jnp.float32), ((0, _NP - _N), (0, 0)))
    zcol = jnp.zeros((_NP,), _f32)

    deg01 = _deg_call(rowp_deg, zcol)
    deg0 = deg01[0].reshape(_NP, 1)
    deg1 = deg01[1].reshape(_NP, 1)

    x0, u, dis1 = _mlp_init(
        xpad, W1.astype(_f32), b1.reshape(1, _D).astype(_f32),
        W2.astype(_f32), b2.reshape(1, _D).astype(_f32), deg0, deg1)

    tx_prev2 = x0
    tx_prev = x0
    outa = None
    for j in range(1, _K + 1):
        s01 = _prop_call(u, rowp, colp)
        if j == 1:
            cc = jnp.stack([coe[0] * 0.5, coe[1]]).reshape(1, 2)
            tx_new, u, outa = _update_first(cc, s01[0], s01[1], dis1, tx_prev2)
        elif j == _K:
            cj = coe[j].reshape(1, 1)
            outa = _update_last(cj, s01[0], s01[1], dis1, tx_prev2, outa)
            break
        else:
            cj = coe[j].reshape(1, 1)
            tx_new, u, outa = _update_rest(
                cj, s01[0], s01[1], dis1, tx_prev2, outa)
        tx_prev2 = tx_prev
        tx_prev = tx_new

    return outa[:_N]


# speedup vs baseline: 1.0413x; 1.0064x over previous
import functools

import jax
import jax.numpy as jnp
import numpy as np
from jax import lax
from jax.experimental import pallas as pl
from jax.experimental.pallas import tpu as pltpu
from jax.experimental.pallas import tpu_sc as plsc

_N = 10000
_E = 320000
_D = 128
_K = 6

_NC = 2
_NS = 16
_NW = _NC * _NS
_NP = 10240
_RPS = _NP // _NS
_CH = 128
_NPH = 2
_NCHP = 40
_NCH = _NPH * _NCHP
_EPAD = _NW * _NCH * _CH


def _cheb_coeff_mat(k):
    j = np.arange(k + 1)
    xj = np.cos((k - j + 0.5) * np.pi / (k + 1))
    t = np.zeros((k + 1, k + 1), dtype=np.float64)
    t[0, :] = 1.0
    t[1, :] = xj
    for i in range(2, k + 1):
        t[i, :] = 2.0 * xj * t[i - 1, :] - t[i - 2, :]
    return (2.0 / (k + 1)) * t


_TMAT = _cheb_coeff_mat(_K).astype(np.float32)

_MESH = plsc.VectorSubcoreMesh(
    core_axis_name="c", subcore_axis_name="s", num_cores=_NC, num_subcores=_NS
)


def _deg_body(row_hbm, zcol_hbm, out_hbm, row_v, ones_v, deg_sh, sem):
    cid = lax.axis_index("c")
    sid = lax.axis_index("s")
    wid = cid * _NS + sid
    sl = pl.ds(sid * _RPS, _RPS)
    pltpu.sync_copy(zcol_hbm.at[sl], deg_sh.at[sl])
    for k in range(_CH // 16):
        ones_v[pl.ds(k * 16, 16)] = jnp.full((16,), 1.0, jnp.float32)
    plsc.subcore_barrier()

    for p in range(_NPH):
        pltpu.sync_copy(row_hbm.at[wid, p], row_v)

        def fire(j, carry):
            pltpu.async_copy(ones_v, deg_sh.at[row_v.at[j]], sem, add=True)
            return carry

        lax.fori_loop(0, _NCHP, fire, 0)

        def drain(j, carry):
            pltpu.make_async_copy(ones_v, deg_sh.at[row_v.at[j]], sem).wait()
            return carry

        lax.fori_loop(0, _NCHP, drain, 0)
    plsc.subcore_barrier()
    pltpu.sync_copy(deg_sh.at[sl], out_hbm.at[cid, sl])


_deg_call = pl.kernel(
    _deg_body,
    out_type=jax.ShapeDtypeStruct((_NC, _NP), jnp.float32),
    mesh=_MESH,
    scratch_types=[
        pltpu.VMEM((_NCHP, _CH), jnp.int32),
        pltpu.VMEM((_CH,), jnp.float32),
        pltpu.VMEM_SHARED((_NP,), jnp.float32),
        pltpu.SemaphoreType.DMA,
    ],
)


def _zero_buf(buf, rows):
    def zbody(r, carry):
        for k in range(_D // 16):
            buf[r, pl.ds(k * 16, 16)] = jnp.full((16,), 0.0, jnp.float32)
        return carry

    lax.fori_loop(0, rows, zbody, 0)


def _prop_body(u_hbm, row_hbm, col_hbm, out_hbm,
               row_v, col_v, buf0, buf1, s_sh, sem0, sem1):
    cid = lax.axis_index("c")
    sid = lax.axis_index("s")
    wid = cid * _NS + sid
    _zero_buf(buf0, _CH)
    for r0 in range(0, _RPS, _CH):
        pltpu.sync_copy(buf0, s_sh.at[pl.ds(sid * _RPS + r0, _CH)])
    plsc.subcore_barrier()

    for p in range(_NPH):
        pltpu.sync_copy(row_hbm.at[wid, p], row_v)
        pltpu.sync_copy(col_hbm.at[wid, p], col_v)

        pltpu.async_copy(u_hbm.at[row_v.at[0]], buf0, sem0)
        pltpu.async_copy(u_hbm.at[row_v.at[1]], buf1, sem1)

        def body(i, carry):
            j = i * 2
            for b, (buf, sem) in enumerate(((buf0, sem0), (buf1, sem1))):
                jb = j + b
                pltpu.make_async_copy(u_hbm.at[row_v.at[jb]], buf, sem).wait()
                pltpu.sync_copy(buf, s_sh.at[col_v.at[jb]], add=True)

                @pl.when(jb + 2 < _NCHP)
                def _():
                    pltpu.async_copy(u_hbm.at[row_v.at[jb + 2]], buf, sem)
            return carry

        lax.fori_loop(0, _NCHP // 2, body, 0)
    plsc.subcore_barrier()
    sl = pl.ds(sid * _RPS, _RPS)
    pltpu.sync_copy(s_sh.at[sl], out_hbm.at[cid, sl])


_prop_call = pl.kernel(
    _prop_body,
    out_type=jax.ShapeDtypeStruct((_NC, _NP, _D), jnp.float32),
    mesh=_MESH,
    scratch_types=[
        pltpu.VMEM((_NCHP, _CH), jnp.int32),
        pltpu.VMEM((_NCHP, _CH), jnp.int32),
        pltpu.VMEM((_CH, _D), jnp.float32),
        pltpu.VMEM((_CH, _D), jnp.float32),
        pltpu.VMEM_SHARED((_NP, _D), jnp.float32),
        pltpu.SemaphoreType.DMA,
        pltpu.SemaphoreType.DMA,
    ],
)


_BR = 5120


def _mlp_init_body(xp_ref, w1_ref, b1_ref, w2_ref, b2_ref,
                   deg0_ref, deg1_ref, x_ref, u_ref, dis1_ref):
    h = jnp.maximum(
        jnp.dot(xp_ref[...], w1_ref[...], preferred_element_type=jnp.float32)
        + b1_ref[...], 0.0)
    x = (jnp.dot(h, w2_ref[...], preferred_element_type=jnp.float32)
         + b2_ref[...])
    deg = deg0_ref[...] + deg1_ref[...]
    dis = jnp.where(deg > 0.0, lax.rsqrt(jnp.maximum(deg, 1e-30)), 0.0)
    x_ref[...] = x
    u_ref[...] = jnp.broadcast_to(dis, x.shape) * x
    dis1_ref[...] = dis


def _update_first_body(c_ref, s0_ref, s1_ref, dis1_ref, txold_ref,
                       tx_ref, u_ref, out_ref):
    dis2 = jnp.broadcast_to(dis1_ref[...], s0_ref.shape)
    tx = -dis2 * (s0_ref[...] + s1_ref[...])
    tx_ref[...] = tx
    u_ref[...] = dis2 * tx
    out_ref[...] = c_ref[0, 0] * txold_ref[...] + c_ref[0, 1] * tx


def _update_rest_body(c_ref, s0_ref, s1_ref, dis1_ref, txold_ref,
                      outold_ref, tx_ref, u_ref, out_ref):
    dis2 = jnp.broadcast_to(dis1_ref[...], s0_ref.shape)
    tx = -2.0 * dis2 * (s0_ref[...] + s1_ref[...]) - txold_ref[...]
    tx_ref[...] = tx
    u_ref[...] = dis2 * tx
    out_ref[...] = outold_ref[...] + c_ref[0, 0] * tx


def _update_last_body(c_ref, s0_ref, s1_ref, dis1_ref, txold_ref,
                      outold_ref, out_ref):
    dis2 = jnp.broadcast_to(dis1_ref[...], s0_ref.shape)
    tx = -2.0 * dis2 * (s0_ref[...] + s1_ref[...]) - txold_ref[...]
    out_ref[...] = outold_ref[...] + c_ref[0, 0] * tx


def _row_spec(i):
    return (i, 0)


def _fixed_spec(i):
    return (0, 0)


_GRID = (_NP // _BR,)
_f32 = jnp.float32


def _mk_mlp_init():
    return pl.pallas_call(
        _mlp_init_body,
        grid=_GRID,
        in_specs=[
            pl.BlockSpec((_BR, _D), _row_spec),
            pl.BlockSpec((_D, _D), _fixed_spec),
            pl.BlockSpec((1, _D), _fixed_spec),
            pl.BlockSpec((_D, _D), _fixed_spec),
            pl.BlockSpec((1, _D), _fixed_spec),
            pl.BlockSpec((_BR, 1), _row_spec),
            pl.BlockSpec((_BR, 1), _row_spec),
        ],
        out_specs=[pl.BlockSpec((_BR, _D), _row_spec)] * 2
        + [pl.BlockSpec((_BR, 1), _row_spec)],
        out_shape=[jax.ShapeDtypeStruct((_NP, _D), _f32)] * 2
        + [jax.ShapeDtypeStruct((_NP, 1), _f32)],
    )


_D_SPEC = pl.BlockSpec((_BR, _D), _row_spec)
_C_SPEC = pl.BlockSpec((_BR, 1), _row_spec)
_S_SPEC = pl.BlockSpec(memory_space=pltpu.SMEM)
_D_SHAPE = jax.ShapeDtypeStruct((_NP, _D), _f32)

_mlp_init = _mk_mlp_init()
_update_first = pl.pallas_call(
    _update_first_body, grid=_GRID,
    in_specs=[_S_SPEC, _D_SPEC, _D_SPEC, _C_SPEC, _D_SPEC],
    out_specs=[_D_SPEC] * 3, out_shape=[_D_SHAPE] * 3)
_update_rest = pl.pallas_call(
    _update_rest_body, grid=_GRID,
    in_specs=[_S_SPEC, _D_SPEC, _D_SPEC, _C_SPEC, _D_SPEC, _D_SPEC],
    out_specs=[_D_SPEC] * 3, out_shape=[_D_SHAPE] * 3)
_update_last = pl.pallas_call(
    _update_last_body, grid=_GRID,
    in_specs=[_S_SPEC, _D_SPEC, _D_SPEC, _C_SPEC, _D_SPEC, _D_SPEC],
    out_specs=_D_SPEC, out_shape=_D_SHAPE)


def kernel(X, edge_index, W1, b1, W2, b2, temp):
    row = edge_index[0].astype(jnp.int32)
    col = edge_index[1].astype(jnp.int32)
    npad = _EPAD - _E
    pad_ar = jnp.arange(npad, dtype=jnp.int32)
    dump = _N + (pad_ar % (_NP - _N))
    realpad = pad_ar % _N
    shp = (_NW, _NPH, _NCHP, _CH)
    rowp = jnp.concatenate([row, realpad]).reshape(shp)
    rowp_deg = jnp.concatenate([row, dump]).reshape(shp)
    colp = jnp.concatenate([col, dump]).reshape(shp)

    coe = jnp.asarray(_TMAT) @ jax.nn.relu(temp.astype(jnp.float32))
    xpad = jnp.pad(X.astype(jnp.float32), ((0, _NP - _N), (0, 0)))
    zcol = jnp.zeros((_NP,), _f32)

    deg01 = _deg_call(rowp_deg, zcol)
    deg0 = deg01[0].reshape(_NP, 1)
    deg1 = deg01[1].reshape(_NP, 1)

    x0, u, dis1 = _mlp_init(
        xpad, W1.astype(_f32), b1.reshape(1, _D).astype(_f32),
        W2.astype(_f32), b2.reshape(1, _D).astype(_f32), deg0, deg1)

    tx_prev2 = x0
    tx_prev = x0
    outa = None
    for j in range(1, _K + 1):
        s01 = _prop_call(u, rowp, colp)
        if j == 1:
            cc = jnp.stack([coe[0] * 0.5, coe[1]]).reshape(1, 2)
            tx_new, u, outa = _update_first(cc, s01[0], s01[1], dis1, tx_prev2)
        elif j == _K:
            cj = coe[j].reshape(1, 1)
            outa = _update_last(cj, s01[0], s01[1], dis1, tx_prev2, outa)
            break
        else:
            cj = coe[j].reshape(1, 1)
            tx_new, u, outa = _update_rest(
                cj, s01[0], s01[1], dis1, tx_prev2, outa)
        tx_prev2 = tx_prev
        tx_prev = tx_new

    return outa[:_N]
